# baseline (speedup 1.0000x reference)
import functools

import jax
import jax.numpy as jnp
from jax.experimental import pallas as pl
from jax.experimental.pallas import tpu as pltpu

_N_SAMPLE = 16


def _agg_kernel(x_ref, neibs_ref, e_ref, wxn_ref, wn_ref, we_ref, wm_ref,
                b_ref, out_ref):
    bn = x_ref.shape[0]
    d_hid = wn_ref.shape[0]

    hx = jax.lax.dot_general(
        x_ref[...], wxn_ref[...],
        (((1,), (1,)), ((), ())), preferred_element_type=jnp.float32)

    hn = jax.lax.dot_general(
        neibs_ref[...], wn_ref[...],
        (((1,), (1,)), ((), ())), preferred_element_type=jnp.float32)
    he = jax.lax.dot_general(
        e_ref[...], we_ref[...],
        (((1,), (1,)), ((), ())), preferred_element_type=jnp.float32)

    a = (hn + he).reshape(bn, _N_SAMPLE, d_hid) + hx[:, None, :]
    m = jnp.max(jax.nn.relu(a), axis=1)

    out_ref[...] = jax.lax.dot_general(
        m.astype(jnp.bfloat16), wm_ref[...],
        (((1,), (1,)), ((), ())), preferred_element_type=jnp.float32
    ) + b_ref[...]


@functools.partial(jax.jit, static_argnames=("block_n",))
def _run(x, neibs, edge_emb, wxn, wn, we, wm, b2d, block_n):
    n, d_in = x.shape
    d_edge = edge_emb.shape[1]
    d_hid = wn.shape[0]
    d_out = wm.shape[0]
    grid = (n // block_n,)

    return pl.pallas_call(
        _agg_kernel,
        grid=grid,
        in_specs=[
            pl.BlockSpec((block_n, d_in), lambda i: (i, 0)),
            pl.BlockSpec((block_n * _N_SAMPLE, d_in), lambda i: (i, 0)),
            pl.BlockSpec((block_n * _N_SAMPLE, d_edge), lambda i: (i, 0)),
            pl.BlockSpec((d_hid, d_in), lambda i: (0, 0)),
            pl.BlockSpec((d_hid, d_in), lambda i: (0, 0)),
            pl.BlockSpec((d_hid, d_edge), lambda i: (0, 0)),
            pl.BlockSpec((d_out, d_hid), lambda i: (0, 0)),
            pl.BlockSpec((1, d_out), lambda i: (0, 0)),
        ],
        out_specs=pl.BlockSpec((block_n, d_out), lambda i: (i, 0)),
        out_shape=jax.ShapeDtypeStruct((n, d_out), jnp.float32),
        compiler_params=pltpu.CompilerParams(
            dimension_semantics=("parallel",)),
    )(x, neibs, edge_emb, wxn, wn, we, wm, b2d)


def kernel(x, neibs, edge_emb, mask, W_x, W_n, W_e, W_m, b_m):
    del mask
    n = x.shape[0]
    wxn = (W_x - W_n).astype(jnp.bfloat16)
    wn = W_n.astype(jnp.bfloat16)
    we = W_e.astype(jnp.bfloat16)
    wm = W_m.astype(jnp.bfloat16)
    b2d = b_m.reshape(1, -1)
    block_n = 400 if n % 400 == 0 else n
    return _run(x.astype(jnp.bfloat16), neibs.astype(jnp.bfloat16),
                edge_emb.astype(jnp.bfloat16), wxn, wn, we, wm, b2d, block_n)


# speedup vs baseline: 1.3054x; 1.3054x over previous
import functools

import jax
import jax.numpy as jnp
from jax.experimental import pallas as pl
from jax.experimental.pallas import tpu as pltpu

_N_SAMPLE = 16


def _agg_kernel(x_ref, neibs_ref, e_ref, wxn_ref, wn_ref, we_ref, wm_ref,
                b_ref, out_ref):
    bn = x_ref.shape[0]
    d_hid = wn_ref.shape[0]

    hx = jax.lax.dot_general(
        x_ref[...].astype(jnp.bfloat16), wxn_ref[...],
        (((1,), (1,)), ((), ())), preferred_element_type=jnp.float32)

    hn = jax.lax.dot_general(
        neibs_ref[...].astype(jnp.bfloat16), wn_ref[...],
        (((1,), (1,)), ((), ())), preferred_element_type=jnp.float32)
    he = jax.lax.dot_general(
        e_ref[...].astype(jnp.bfloat16), we_ref[...],
        (((1,), (1,)), ((), ())), preferred_element_type=jnp.float32)

    a = (hn + he).reshape(bn, _N_SAMPLE, d_hid) + hx[:, None, :]
    m = jnp.max(jax.nn.relu(a), axis=1)

    out_ref[...] = jax.lax.dot_general(
        m.astype(jnp.bfloat16), wm_ref[...],
        (((1,), (1,)), ((), ())), preferred_element_type=jnp.float32
    ) + b_ref[...]


@functools.partial(jax.jit, static_argnames=("block_n",))
def _run(x, neibs, edge_emb, wxn, wn, we, wm, b2d, block_n):
    n, d_in = x.shape
    d_edge = edge_emb.shape[1]
    d_hid = wn.shape[0]
    d_out = wm.shape[0]
    grid = (n // block_n,)

    return pl.pallas_call(
        _agg_kernel,
        grid=grid,
        in_specs=[
            pl.BlockSpec((block_n, d_in), lambda i: (i, 0)),
            pl.BlockSpec((block_n * _N_SAMPLE, d_in), lambda i: (i, 0)),
            pl.BlockSpec((block_n * _N_SAMPLE, d_edge), lambda i: (i, 0)),
            pl.BlockSpec((d_hid, d_in), lambda i: (0, 0)),
            pl.BlockSpec((d_hid, d_in), lambda i: (0, 0)),
            pl.BlockSpec((d_hid, d_edge), lambda i: (0, 0)),
            pl.BlockSpec((d_out, d_hid), lambda i: (0, 0)),
            pl.BlockSpec((1, d_out), lambda i: (0, 0)),
        ],
        out_specs=pl.BlockSpec((block_n, d_out), lambda i: (i, 0)),
        out_shape=jax.ShapeDtypeStruct((n, d_out), jnp.float32),
        compiler_params=pltpu.CompilerParams(
            dimension_semantics=("parallel",)),
    )(x, neibs, edge_emb, wxn, wn, we, wm, b2d)


def kernel(x, neibs, edge_emb, mask, W_x, W_n, W_e, W_m, b_m):
    del mask
    n = x.shape[0]
    wxn = (W_x - W_n).astype(jnp.bfloat16)
    wn = W_n.astype(jnp.bfloat16)
    we = W_e.astype(jnp.bfloat16)
    wm = W_m.astype(jnp.bfloat16)
    b2d = b_m.reshape(1, -1)
    block_n = 400 if n % 400 == 0 else n
    return _run(x, neibs, edge_emb, wxn, wn, we, wm, b2d, block_n)


# speedup vs baseline: 1.3335x; 1.0215x over previous
import functools

import jax
import jax.numpy as jnp
from jax.experimental import pallas as pl
from jax.experimental.pallas import tpu as pltpu

_N_SAMPLE = 16


def _agg_kernel(x_ref, neibs_ref, e_ref, wxn_ref, wn_ref, we_ref, wm_ref,
                b_ref, out_ref):
    bn = x_ref.shape[0]
    d_hid = wn_ref.shape[0]

    hx = jax.lax.dot_general(
        x_ref[...].astype(jnp.bfloat16), wxn_ref[...],
        (((1,), (1,)), ((), ())), preferred_element_type=jnp.float32)

    hn = jax.lax.dot_general(
        neibs_ref[...].astype(jnp.bfloat16), wn_ref[...],
        (((1,), (1,)), ((), ())), preferred_element_type=jnp.float32)
    he = jax.lax.dot_general(
        e_ref[...].astype(jnp.bfloat16), we_ref[...],
        (((1,), (1,)), ((), ())), preferred_element_type=jnp.float32)

    a = (hn + he).reshape(bn, _N_SAMPLE, d_hid)
    m = jax.nn.relu(jnp.max(a, axis=1) + hx)

    out_ref[...] = jax.lax.dot_general(
        m.astype(jnp.bfloat16), wm_ref[...],
        (((1,), (1,)), ((), ())), preferred_element_type=jnp.float32
    ) + b_ref[...]


@functools.partial(jax.jit, static_argnames=("block_n",))
def _run(x, neibs, edge_emb, wxn, wn, we, wm, b2d, block_n):
    n, d_in = x.shape
    d_edge = edge_emb.shape[1]
    d_hid = wn.shape[0]
    d_out = wm.shape[0]
    grid = (n // block_n,)

    return pl.pallas_call(
        _agg_kernel,
        grid=grid,
        in_specs=[
            pl.BlockSpec((block_n, d_in), lambda i: (i, 0)),
            pl.BlockSpec((block_n * _N_SAMPLE, d_in), lambda i: (i, 0)),
            pl.BlockSpec((block_n * _N_SAMPLE, d_edge), lambda i: (i, 0)),
            pl.BlockSpec((d_hid, d_in), lambda i: (0, 0)),
            pl.BlockSpec((d_hid, d_in), lambda i: (0, 0)),
            pl.BlockSpec((d_hid, d_edge), lambda i: (0, 0)),
            pl.BlockSpec((d_out, d_hid), lambda i: (0, 0)),
            pl.BlockSpec((1, d_out), lambda i: (0, 0)),
        ],
        out_specs=pl.BlockSpec((block_n, d_out), lambda i: (i, 0)),
        out_shape=jax.ShapeDtypeStruct((n, d_out), jnp.float32),
        compiler_params=pltpu.CompilerParams(
            dimension_semantics=("parallel",)),
    )(x, neibs, edge_emb, wxn, wn, we, wm, b2d)


def kernel(x, neibs, edge_emb, mask, W_x, W_n, W_e, W_m, b_m):
    del mask
    n = x.shape[0]
    wxn = (W_x - W_n).astype(jnp.bfloat16)
    wn = W_n.astype(jnp.bfloat16)
    we = W_e.astype(jnp.bfloat16)
    wm = W_m.astype(jnp.bfloat16)
    b2d = b_m.reshape(1, -1)
    block_n = 400 if n % 400 == 0 else n
    return _run(x, neibs, edge_emb, wxn, wn, we, wm, b2d, block_n)
